# baseline (speedup 1.0000x reference)
import functools

import jax
import jax.numpy as jnp
from jax.experimental import pallas as pl
from jax.experimental.pallas import tpu as pltpu

N = 10000
E = 320000
D = 128
BE = 512
NB = E // BE


def _edge1_body(ef_ref, gsum_ref, w1_ref, wp_ref, attn_ref, logits_ref, f2pre_ref):
    ef = ef_ref[...]
    f1 = jnp.dot(ef, w1_ref[...], preferred_element_type=jnp.float32)
    u = f1 + gsum_ref[...]
    fo = jnp.where(u > 0, u, 0.01 * u)
    lg = jnp.sum(fo * attn_ref[...], axis=1)
    logits_ref[...] = lg.reshape(1, 1, BE)
    h1 = jnp.maximum(fo, 0.0) + ef
    f2pre_ref[...] = jnp.dot(h1, wp_ref[...], preferred_element_type=jnp.float32)


def _edge1(ef, gsum, w1, wp, attn):
    logits3, f2pre = pl.pallas_call(
        _edge1_body,
        grid=(NB,),
        in_specs=[
            pl.BlockSpec((BE, D), lambda i: (i, 0)),
            pl.BlockSpec((BE, D), lambda i: (i, 0)),
            pl.BlockSpec((D, D), lambda i: (0, 0)),
            pl.BlockSpec((D, D), lambda i: (0, 0)),
            pl.BlockSpec((1, D), lambda i: (0, 0)),
        ],
        out_specs=[
            pl.BlockSpec((1, 1, BE), lambda i: (i, 0, 0)),
            pl.BlockSpec((BE, D), lambda i: (i, 0)),
        ],
        out_shape=[
            jax.ShapeDtypeStruct((NB, 1, BE), jnp.float32),
            jax.ShapeDtypeStruct((E, D), jnp.float32),
        ],
    )(ef, gsum, w1, wp, attn)
    return logits3.reshape(E), f2pre


def _edge2_body(f2pre_ref, gsum_ref, attn_ref, logits_ref):
    u = f2pre_ref[...] + gsum_ref[...]
    fo = jnp.where(u > 0, u, 0.01 * u)
    lg = jnp.sum(fo * attn_ref[...], axis=1)
    logits_ref[...] = lg.reshape(1, 1, BE)


def _edge2(f2pre, gsum, attn):
    logits3 = pl.pallas_call(
        _edge2_body,
        grid=(NB,),
        in_specs=[
            pl.BlockSpec((BE, D), lambda i: (i, 0)),
            pl.BlockSpec((BE, D), lambda i: (i, 0)),
            pl.BlockSpec((1, D), lambda i: (0, 0)),
        ],
        out_specs=pl.BlockSpec((1, 1, BE), lambda i: (i, 0, 0)),
        out_shape=jax.ShapeDtypeStruct((NB, 1, BE), jnp.float32),
    )(f2pre, gsum, attn)
    return logits3.reshape(E)


def _softmax_weights(logits, dst):
    m = jax.ops.segment_max(logits, dst, num_segments=N)
    m = jnp.where(jnp.isfinite(m), m, 0.0)
    ex = jnp.exp(logits - m[dst])
    denom = jax.ops.segment_sum(ex, dst, num_segments=N)
    return ex / denom[dst]


def kernel(node_feats, edge_feats, edge_index, W_ni1, W_nj1, W_fij1, attn1,
           W_node1, b_node1, W_ni2, W_nj2, W_fij2, attn2, W_node2, b_node2):
    x = node_feats
    ef = edge_feats
    src = edge_index[0]
    dst = edge_index[1]

    f_ni1 = x @ W_ni1
    f_nj1 = x @ W_nj1
    hn1 = x @ W_node1 + b_node1[None, :]

    gsum1 = f_ni1[src] + f_nj1[dst]
    logits1, f2pre = _edge1(ef, gsum1, W_fij1, W_fij2, attn1)

    a1 = _softmax_weights(logits1, dst)
    out1 = jax.ops.segment_sum(hn1[src] * a1[:, None], dst, num_segments=N)
    h0 = jax.nn.relu(out1) + x

    f_ni2 = h0 @ W_ni2
    f_nj2 = h0 @ W_nj2
    hn2 = h0 @ W_node2 + b_node2[None, :]

    gsum2 = f_ni2[src] + f_nj2[dst]
    logits2 = _edge2(f2pre, gsum2, attn2)

    a2 = _softmax_weights(logits2, dst)
    out2 = jax.ops.segment_sum(hn2[src] * a2[:, None], dst, num_segments=N)
    return out2 + x


# speedup vs baseline: 5.3958x; 5.3958x over previous
import functools

import jax
import jax.numpy as jnp
from jax import lax
from jax.experimental import pallas as pl
from jax.experimental.pallas import tpu as pltpu
from jax.experimental.pallas import tpu_sc as plsc

N = 10000
E = 320000
D = 128
BE = 512
NB = E // BE

NC = 2
NS = 16
NW = NC * NS
CHUNK = 128
NCHUNK = E // CHUNK
NP = 10240
SL = NP // NS


def _sc_mesh():
    return plsc.VectorSubcoreMesh(core_axis_name="c", subcore_axis_name="s")


def _gsum_sc(fni, fnj, src, dst):
    @functools.partial(
        pl.kernel,
        mesh=_sc_mesh(),
        out_type=jax.ShapeDtypeStruct((E, D), jnp.float32),
        scratch_types=[
            pltpu.VMEM((CHUNK,), jnp.int32),
            pltpu.VMEM((CHUNK,), jnp.int32),
            pltpu.VMEM((CHUNK, D), jnp.float32),
            pltpu.VMEM((CHUNK, D), jnp.float32),
            pltpu.SemaphoreType.DMA,
            pltpu.SemaphoreType.DMA,
        ],
    )
    def k(fni_hbm, fnj_hbm, src_hbm, dst_hbm, out_hbm, src_v, dst_v, r1, r2,
          sem1, sem2):
        wid = lax.axis_index("s") * NC + lax.axis_index("c")
        nk = NCHUNK // NW + jnp.where(wid < (NCHUNK % NW), 1, 0)

        def chunk_body(j, carry):
            base = (wid + j * NW) * CHUNK
            pltpu.sync_copy(src_hbm.at[pl.ds(base, CHUNK)], src_v)
            pltpu.sync_copy(dst_hbm.at[pl.ds(base, CHUNK)], dst_v)
            cp1 = pltpu.async_copy(fni_hbm.at[src_v], r1, sem1)
            cp2 = pltpu.async_copy(fnj_hbm.at[dst_v], r2, sem2)
            cp1.wait()
            cp2.wait()

            def add_row(r, c2):
                for d8 in range(D // 16):
                    sl = pl.ds(d8 * 16, 16)
                    r1[r, sl] = r1[r, sl] + r2[r, sl]
                return c2

            lax.fori_loop(0, CHUNK, add_row, 0)
            pltpu.sync_copy(r1, out_hbm.at[pl.ds(base, CHUNK)])
            return carry

        lax.fori_loop(0, nk, chunk_body, 0)

    return k(fni, fnj, src, dst)


def _denom_sc(logits, dst, mvec):
    @functools.partial(
        pl.kernel,
        mesh=_sc_mesh(),
        out_type=jax.ShapeDtypeStruct((NC, NP), jnp.float32),
        scratch_types=[
            pltpu.VMEM((CHUNK,), jnp.float32),
            pltpu.VMEM((CHUNK,), jnp.int32),
            pltpu.VMEM((CHUNK,), jnp.float32),
            pltpu.VMEM((16,), jnp.float32),
            pltpu.VMEM((SL,), jnp.float32),
            pltpu.VMEM_SHARED((NP,), jnp.float32),
        ],
    )
    def k(lg_hbm, dst_hbm, mv_hbm, out_hbm, lg_v, dst_v, ex_v, m_v, z_v,
          den_sh):
        cid = lax.axis_index("c")
        sid = lax.axis_index("s")
        wid = sid * NC + cid

        def zset(r, c):
            z_v[pl.ds(r * 16, 16)] = jnp.zeros((16,), jnp.float32)
            return c

        lax.fori_loop(0, SL // 16, zset, 0)
        pltpu.sync_copy(z_v, den_sh.at[pl.ds(sid * SL, SL)])
        plsc.subcore_barrier()

        pltpu.sync_copy(mv_hbm, m_v)
        mvec = m_v[...]

        nk = NCHUNK // NW + jnp.where(wid < (NCHUNK % NW), 1, 0)

        def chunk_body(j, carry):
            base = (wid + j * NW) * CHUNK
            pltpu.sync_copy(lg_hbm.at[pl.ds(base, CHUNK)], lg_v)
            pltpu.sync_copy(dst_hbm.at[pl.ds(base, CHUNK)], dst_v)
            for g in range(CHUNK // 16):
                sl = pl.ds(g * 16, 16)
                ex_v[sl] = jnp.exp(lg_v[sl] - mvec)
            pltpu.sync_copy(ex_v, den_sh.at[dst_v], add=True)
            return carry

        lax.fori_loop(0, nk, chunk_body, 0)
        plsc.subcore_barrier()
        pltpu.sync_copy(den_sh.at[pl.ds(sid * SL, SL)],
                        out_hbm.at[cid, pl.ds(sid * SL, SL)])

    return k(logits, dst, mvec)


_GDNUMS = lax.GatherDimensionNumbers(
    offset_dims=(), collapsed_slice_dims=(0,), start_index_map=(0,))


def _bcast_lane(v16, i):
    idx = jnp.full((16, 1), i, jnp.int32)
    return lax.gather(v16, idx, _GDNUMS, (1,),
                      mode=lax.GatherScatterMode.PROMISE_IN_BOUNDS)


def _agg_sc(logits, src, dst, mvec, rec, hn):
    @functools.partial(
        pl.kernel,
        mesh=_sc_mesh(),
        out_type=jax.ShapeDtypeStruct((NC, NP, D), jnp.float32),
        scratch_types=[
            pltpu.VMEM((CHUNK,), jnp.float32),
            pltpu.VMEM((CHUNK,), jnp.int32),
            pltpu.VMEM((CHUNK,), jnp.int32),
            pltpu.VMEM((CHUNK,), jnp.float32),
            pltpu.VMEM((16,), jnp.float32),
            pltpu.VMEM((CHUNK, D), jnp.float32),
            pltpu.VMEM_SHARED((NP, D), jnp.float32),
            pltpu.SemaphoreType.DMA,
            pltpu.SemaphoreType.DMA,
        ],
    )
    def k(lg_hbm, src_hbm, dst_hbm, mv_hbm, rec_hbm, hn_hbm, out_hbm,
          lg_v, src_v, dst_v, rec_g, m_v, rows, acc_sh, sem, sem2):
        cid = lax.axis_index("c")
        sid = lax.axis_index("s")
        wid = sid * NC + cid

        def zrow(r, c):
            for d8 in range(D // 16):
                rows[r, pl.ds(d8 * 16, 16)] = jnp.zeros((16,), jnp.float32)
            return c

        lax.fori_loop(0, CHUNK, zrow, 0)
        for t in range(SL // CHUNK):
            pltpu.sync_copy(rows, acc_sh.at[pl.ds(sid * SL + t * CHUNK, CHUNK)])
        plsc.subcore_barrier()

        pltpu.sync_copy(mv_hbm, m_v)
        mvec = m_v[...]

        nk = NCHUNK // NW + jnp.where(wid < (NCHUNK % NW), 1, 0)

        def chunk_body(j, carry):
            base = (wid + j * NW) * CHUNK
            pltpu.sync_copy(lg_hbm.at[pl.ds(base, CHUNK)], lg_v)
            pltpu.sync_copy(src_hbm.at[pl.ds(base, CHUNK)], src_v)
            pltpu.sync_copy(dst_hbm.at[pl.ds(base, CHUNK)], dst_v)
            cp_rows = pltpu.async_copy(hn_hbm.at[src_v], rows, sem)
            cp_rec = pltpu.async_copy(rec_hbm.at[dst_v], rec_g, sem2)
            cp_rec.wait()
            cp_rows.wait()

            for g in range(CHUNK // 16):
                sl = pl.ds(g * 16, 16)
                a16 = jnp.exp(lg_v[sl] - mvec) * rec_g[sl]
                for i in range(16):
                    r = g * 16 + i
                    bc = _bcast_lane(a16, i)
                    for d8 in range(D // 16):
                        sl2 = pl.ds(d8 * 16, 16)
                        rows[r, sl2] = rows[r, sl2] * bc

            pltpu.sync_copy(rows, acc_sh.at[dst_v], add=True)
            return carry

        lax.fori_loop(0, nk, chunk_body, 0)
        plsc.subcore_barrier()
        pltpu.sync_copy(acc_sh.at[pl.ds(sid * SL, SL)],
                        out_hbm.at[cid, pl.ds(sid * SL, SL)])

    return k(logits, src, dst, mvec, rec, hn)


def _edge1_body(ef_ref, gsum_ref, w1_ref, wp_ref, attn_ref, logits_ref,
                f2pre_ref, mx_ref):
    i = pl.program_id(0)

    @pl.when(i == 0)
    def _init():
        mx_ref[...] = jnp.full((1, D), -1e30, jnp.float32)

    ef = ef_ref[...]
    f1 = jnp.dot(ef, w1_ref[...], preferred_element_type=jnp.float32)
    u = f1 + gsum_ref[...]
    fo = jnp.where(u > 0, u, 0.01 * u)
    lg = jnp.sum(fo * attn_ref[...], axis=1)
    logits_ref[...] = lg.reshape(1, 1, BE)
    cur = jnp.broadcast_to(jnp.max(lg), (1, D))
    mx_ref[...] = jnp.maximum(mx_ref[...], cur)
    h1 = jnp.maximum(fo, 0.0) + ef
    f2pre_ref[...] = jnp.dot(h1, wp_ref[...], preferred_element_type=jnp.float32)


def _edge1(ef, gsum, w1, wp, attn):
    logits3, f2pre, mx = pl.pallas_call(
        _edge1_body,
        grid=(NB,),
        in_specs=[
            pl.BlockSpec((BE, D), lambda i: (i, 0)),
            pl.BlockSpec((BE, D), lambda i: (i, 0)),
            pl.BlockSpec((D, D), lambda i: (0, 0)),
            pl.BlockSpec((D, D), lambda i: (0, 0)),
            pl.BlockSpec((1, D), lambda i: (0, 0)),
        ],
        out_specs=[
            pl.BlockSpec((1, 1, BE), lambda i: (i, 0, 0)),
            pl.BlockSpec((BE, D), lambda i: (i, 0)),
            pl.BlockSpec((1, D), lambda i: (0, 0)),
        ],
        out_shape=[
            jax.ShapeDtypeStruct((NB, 1, BE), jnp.float32),
            jax.ShapeDtypeStruct((E, D), jnp.float32),
            jax.ShapeDtypeStruct((1, D), jnp.float32),
        ],
    )(ef, gsum, w1, wp, attn)
    return logits3.reshape(E), f2pre, mx


def _edge2_body(f2pre_ref, gsum_ref, attn_ref, logits_ref, mx_ref):
    i = pl.program_id(0)

    @pl.when(i == 0)
    def _init():
        mx_ref[...] = jnp.full((1, D), -1e30, jnp.float32)

    u = f2pre_ref[...] + gsum_ref[...]
    fo = jnp.where(u > 0, u, 0.01 * u)
    lg = jnp.sum(fo * attn_ref[...], axis=1)
    logits_ref[...] = lg.reshape(1, 1, BE)
    cur = jnp.broadcast_to(jnp.max(lg), (1, D))
    mx_ref[...] = jnp.maximum(mx_ref[...], cur)


def _edge2(f2pre, gsum, attn):
    logits3, mx = pl.pallas_call(
        _edge2_body,
        grid=(NB,),
        in_specs=[
            pl.BlockSpec((BE, D), lambda i: (i, 0)),
            pl.BlockSpec((BE, D), lambda i: (i, 0)),
            pl.BlockSpec((1, D), lambda i: (0, 0)),
        ],
        out_specs=[
            pl.BlockSpec((1, 1, BE), lambda i: (i, 0, 0)),
            pl.BlockSpec((1, D), lambda i: (0, 0)),
        ],
        out_shape=[
            jax.ShapeDtypeStruct((NB, 1, BE), jnp.float32),
            jax.ShapeDtypeStruct((1, D), jnp.float32),
        ],
    )(f2pre, gsum, attn)
    return logits3.reshape(E), mx


BRN = 1000
NBN = N // BRN


def _node1_body(x_ref, wni_ref, wnj_ref, wnode_ref, b_ref,
                fni_ref, fnj_ref, hn_ref):
    x = x_ref[...]
    fni_ref[...] = jnp.dot(x, wni_ref[...], preferred_element_type=jnp.float32)
    fnj_ref[...] = jnp.dot(x, wnj_ref[...], preferred_element_type=jnp.float32)
    hn_ref[...] = (jnp.dot(x, wnode_ref[...], preferred_element_type=jnp.float32)
                   + b_ref[...])


def _node_tables1(x, wni, wnj, wnode, b):
    return pl.pallas_call(
        _node1_body,
        grid=(NBN,),
        in_specs=[
            pl.BlockSpec((BRN, D), lambda i: (i, 0)),
            pl.BlockSpec((D, D), lambda i: (0, 0)),
            pl.BlockSpec((D, D), lambda i: (0, 0)),
            pl.BlockSpec((D, D), lambda i: (0, 0)),
            pl.BlockSpec((1, D), lambda i: (0, 0)),
        ],
        out_specs=[
            pl.BlockSpec((BRN, D), lambda i: (i, 0)),
            pl.BlockSpec((BRN, D), lambda i: (i, 0)),
            pl.BlockSpec((BRN, D), lambda i: (i, 0)),
        ],
        out_shape=[jax.ShapeDtypeStruct((N, D), jnp.float32)] * 3,
    )(x, wni, wnj, wnode, b.reshape(1, D))


def _node2_body(accp_ref, x_ref, wni_ref, wnj_ref, wnode_ref, b_ref,
                fni_ref, fnj_ref, hn_ref):
    out1 = accp_ref[0] + accp_ref[1]
    h0 = jnp.maximum(out1, 0.0) + x_ref[...]
    fni_ref[...] = jnp.dot(h0, wni_ref[...], preferred_element_type=jnp.float32)
    fnj_ref[...] = jnp.dot(h0, wnj_ref[...], preferred_element_type=jnp.float32)
    hn_ref[...] = (jnp.dot(h0, wnode_ref[...], preferred_element_type=jnp.float32)
                   + b_ref[...])


def _node_tables2(accp, x, wni, wnj, wnode, b):
    return pl.pallas_call(
        _node2_body,
        grid=(NBN,),
        in_specs=[
            pl.BlockSpec((NC, BRN, D), lambda i: (0, i, 0)),
            pl.BlockSpec((BRN, D), lambda i: (i, 0)),
            pl.BlockSpec((D, D), lambda i: (0, 0)),
            pl.BlockSpec((D, D), lambda i: (0, 0)),
            pl.BlockSpec((D, D), lambda i: (0, 0)),
            pl.BlockSpec((1, D), lambda i: (0, 0)),
        ],
        out_specs=[
            pl.BlockSpec((BRN, D), lambda i: (i, 0)),
            pl.BlockSpec((BRN, D), lambda i: (i, 0)),
            pl.BlockSpec((BRN, D), lambda i: (i, 0)),
        ],
        out_shape=[jax.ShapeDtypeStruct((N, D), jnp.float32)] * 3,
    )(accp, x, wni, wnj, wnode, b.reshape(1, D))


def _final_body(accp_ref, x_ref, out_ref):
    out_ref[...] = accp_ref[0] + accp_ref[1] + x_ref[...]


def _final(accp, x):
    return pl.pallas_call(
        _final_body,
        grid=(NBN,),
        in_specs=[
            pl.BlockSpec((NC, BRN, D), lambda i: (0, i, 0)),
            pl.BlockSpec((BRN, D), lambda i: (i, 0)),
        ],
        out_specs=pl.BlockSpec((BRN, D), lambda i: (i, 0)),
        out_shape=jax.ShapeDtypeStruct((N, D), jnp.float32),
    )(accp, x)


def _egat_sparse(logits, mx, src, dst, hn):
    mvec = mx[0, :16]
    denp = _denom_sc(logits, dst, mvec)
    rec = 1.0 / (denp[0] + denp[1])
    return _agg_sc(logits, src, dst, mvec, rec, hn)


def kernel(node_feats, edge_feats, edge_index, W_ni1, W_nj1, W_fij1, attn1,
           W_node1, b_node1, W_ni2, W_nj2, W_fij2, attn2, W_node2, b_node2):
    x = node_feats
    ef = edge_feats
    src = edge_index[0]
    dst = edge_index[1]

    f_ni1, f_nj1, hn1 = _node_tables1(x, W_ni1, W_nj1, W_node1, b_node1)
    gsum1 = _gsum_sc(f_ni1, f_nj1, src, dst)
    logits1, f2pre, mx1 = _edge1(ef, gsum1, W_fij1, W_fij2, attn1)
    accp1 = _egat_sparse(logits1, mx1, src, dst, hn1)

    f_ni2, f_nj2, hn2 = _node_tables2(accp1, x, W_ni2, W_nj2, W_node2, b_node2)
    gsum2 = _gsum_sc(f_ni2, f_nj2, src, dst)
    logits2, mx2 = _edge2(f2pre, gsum2, attn2)
    accp2 = _egat_sparse(logits2, mx2, src, dst, hn2)

    return _final(accp2, x)


# speedup vs baseline: 7.6417x; 1.4162x over previous
import functools

import jax
import jax.numpy as jnp
from jax import lax
from jax.experimental import pallas as pl
from jax.experimental.pallas import tpu as pltpu
from jax.experimental.pallas import tpu_sc as plsc

N = 10000
E = 320000
D = 128
BE = 512
NB = E // BE

NC = 2
NS = 16
NW = NC * NS
CHUNK = 128
NCHUNK = E // CHUNK
NP = 10240
SL = NP // NS

CG = 80
EW = E // NW
NKG = EW // CG
EW2 = E // NS
NKD = EW2 // CG


def _sc_mesh():
    return plsc.VectorSubcoreMesh(core_axis_name="c", subcore_axis_name="s")


def _gsum_sc(fni, fnj, src2d, dst2d):
    @functools.partial(
        pl.kernel,
        mesh=_sc_mesh(),
        out_type=jax.ShapeDtypeStruct((E, D), jnp.float32),
        scratch_types=[
            pltpu.VMEM((NKG, CG), jnp.int32),
            pltpu.VMEM((NKG, CG), jnp.int32),
            pltpu.VMEM((CG, D), jnp.float32),
            pltpu.VMEM((CG, D), jnp.float32),
            pltpu.VMEM((CG, D), jnp.float32),
            pltpu.VMEM((CG, D), jnp.float32),
            pltpu.VMEM((CG, D), jnp.float32),
            pltpu.VMEM((CG, D), jnp.float32),
            pltpu.SemaphoreType.DMA,
            pltpu.SemaphoreType.DMA,
            pltpu.SemaphoreType.DMA,
            pltpu.SemaphoreType.DMA,
        ],
    )
    def k(fni_hbm, fnj_hbm, src_hbm, dst_hbm, out_hbm,
          src_t, dst_t, r1a, r2a, r1b, r2b, oa, ob, ga, gb, sa, sb):
        wid = lax.axis_index("s") * NC + lax.axis_index("c")
        row0 = wid * NKG
        pltpu.sync_copy(src_hbm.at[wid], src_t)
        pltpu.sync_copy(dst_hbm.at[wid], dst_t)

        bufs = ((r1a, r2a, oa, ga, sa), (r1b, r2b, ob, gb, sb))

        def issue(c, b):
            r1, r2, _, g, _ = bufs[b]
            pltpu.async_copy(fni_hbm.at[src_t.at[c]], r1, g)
            pltpu.async_copy(fnj_hbm.at[dst_t.at[c]], r2, g)

        def wait_g(b):
            r1, r2, _, g, _ = bufs[b]
            pltpu.make_async_copy(fni_hbm.at[src_t.at[0]], r1, g).wait()
            pltpu.make_async_copy(fnj_hbm.at[dst_t.at[0]], r2, g).wait()

        def compute(c, b):
            r1, r2, o, _, s = bufs[b]

            def add_row(r, cy):
                for d8 in range(D // 16):
                    sl = pl.ds(d8 * 16, 16)
                    o[r, sl] = r1[r, sl] + r2[r, sl]
                return cy

            lax.fori_loop(0, CG, add_row, 0)
            pltpu.async_copy(o, out_hbm.at[pl.ds((row0 + c) * CG, CG)], s)

        def wait_o(b):
            _, _, o, _, s = bufs[b]
            pltpu.make_async_copy(o, out_hbm.at[pl.ds(0, CG)], s).wait()

        issue(0, 0)

        def body(j, cy):
            c0 = 2 * j
            issue(c0 + 1, 1)
            wait_g(0)

            @pl.when(j > 0)
            def _():
                wait_o(0)

            compute(c0, 0)
            issue(c0 + 2, 0)
            wait_g(1)

            @pl.when(j > 0)
            def _():
                wait_o(1)

            compute(c0 + 1, 1)
            return cy

        lax.fori_loop(0, (NKG - 1) // 2, body, 0)
        wait_g(0)
        wait_o(0)
        compute(NKG - 1, 0)
        wait_o(0)
        wait_o(1)

    return k(fni, fnj, src2d, dst2d)


def _denom_sc(lg2d, dst2d, mvec):
    WAVE = 10

    @functools.partial(
        pl.kernel,
        mesh=_sc_mesh(),
        out_type=jax.ShapeDtypeStruct((NC, NP), jnp.float32),
        scratch_types=[
            pltpu.VMEM((NKD, CG), jnp.float32),
            pltpu.VMEM((NKD, CG), jnp.int32),
            pltpu.VMEM((NKD, CG), jnp.float32),
            pltpu.VMEM((16,), jnp.float32),
            pltpu.VMEM((SL,), jnp.float32),
            pltpu.VMEM_SHARED((NP,), jnp.float32),
            pltpu.SemaphoreType.DMA,
        ],
    )
    def k(lg_hbm, dst_hbm, mv_hbm, out_hbm, lg_t, dst_t, ex_t, m_v, z_v,
          den_sh, sem):
        cid = lax.axis_index("c")
        sid = lax.axis_index("s")
        row0 = sid * NKD

        def zset(r, c):
            z_v[pl.ds(r * 16, 16)] = jnp.zeros((16,), jnp.float32)
            return c

        lax.fori_loop(0, SL // 16, zset, 0)
        pltpu.sync_copy(z_v, den_sh.at[pl.ds(sid * SL, SL)])

        pltpu.sync_copy(lg_hbm.at[sid], lg_t)
        pltpu.sync_copy(dst_hbm.at[sid], dst_t)
        pltpu.sync_copy(mv_hbm, m_v)
        mvec = m_v[...]

        def exp_chunk(c, cy):
            for g in range(CG // 16):
                sl = pl.ds(g * 16, 16)
                ex_t[c, sl] = jnp.exp(lg_t[c, sl] - mvec)
            return cy

        lax.fori_loop(0, NKD, exp_chunk, 0)
        plsc.subcore_barrier()

        def wave(j, cy):
            for w in range(WAVE):
                c = j * WAVE + w
                pltpu.async_copy(ex_t.at[c], den_sh.at[dst_t.at[c]], sem,
                                 add=True)
            for w in range(WAVE):
                pltpu.make_async_copy(ex_t.at[0], den_sh.at[dst_t.at[0]],
                                      sem).wait()
            return cy

        lax.fori_loop(0, NKD // WAVE, wave, 0)
        plsc.subcore_barrier()
        pltpu.sync_copy(den_sh.at[pl.ds(sid * SL, SL)],
                        out_hbm.at[cid, pl.ds(sid * SL, SL)])

    return k(lg2d, dst2d, mvec)


_GDNUMS = lax.GatherDimensionNumbers(
    offset_dims=(), collapsed_slice_dims=(0,), start_index_map=(0,))


def _bcast_lane(v16, i):
    idx = jnp.full((16, 1), i, jnp.int32)
    return lax.gather(v16, idx, _GDNUMS, (1,),
                      mode=lax.GatherScatterMode.PROMISE_IN_BOUNDS)


def _agg_sc(logits, src, dst, mvec, rec, hn):
    @functools.partial(
        pl.kernel,
        mesh=_sc_mesh(),
        out_type=jax.ShapeDtypeStruct((NC, NP, D), jnp.float32),
        scratch_types=[
            pltpu.VMEM((NKG, CG), jnp.int32),
            pltpu.VMEM((16,), jnp.float32),
            pltpu.VMEM((CG, D), jnp.float32),
            pltpu.VMEM((CG, D), jnp.float32),
            pltpu.VMEM((CG,), jnp.float32),
            pltpu.VMEM((CG,), jnp.float32),
            pltpu.VMEM((CG,), jnp.float32),
            pltpu.VMEM((CG,), jnp.float32),
            pltpu.VMEM((CG,), jnp.int32),
            pltpu.VMEM((CG,), jnp.int32),
            pltpu.VMEM_SHARED((NP, D), jnp.float32),
            pltpu.SemaphoreType.DMA,
            pltpu.SemaphoreType.DMA,
        ],
    )
    def k(lg_hbm, src_hbm, dst_hbm, mv_hbm, rec_hbm, hn_hbm, out_hbm,
          dst_t, m_v, rowsa, rowsb, reca, recb, lga, lgb, srca, srcb,
          acc_sh, ga, gb):
        cid = lax.axis_index("c")
        sid = lax.axis_index("s")
        wid = sid * NC + cid

        def zrow(r, c):
            for d8 in range(D // 16):
                rowsa[r, pl.ds(d8 * 16, 16)] = jnp.zeros((16,), jnp.float32)
            return c

        lax.fori_loop(0, CG, zrow, 0)
        for t in range(SL // CG):
            pltpu.sync_copy(rowsa, acc_sh.at[pl.ds(sid * SL + t * CG, CG)])
        plsc.subcore_barrier()

        pltpu.sync_copy(dst_hbm.at[wid], dst_t)
        pltpu.sync_copy(mv_hbm, m_v)
        mvec = m_v[...]

        bufs = ((rowsa, reca, lga, srca, ga), (rowsb, recb, lgb, srcb, gb))

        def issue(c, b):
            rows, rec_g, lg_c, src_c, g = bufs[b]
            pltpu.sync_copy(src_hbm.at[wid, c], src_c)
            pltpu.async_copy(hn_hbm.at[src_c], rows, g)
            pltpu.async_copy(rec_hbm.at[dst_t.at[c]], rec_g, g)
            pltpu.async_copy(lg_hbm.at[wid, c], lg_c, g)

        def wait_g(b):
            rows, rec_g, lg_c, src_c, g = bufs[b]
            pltpu.make_async_copy(hn_hbm.at[src_c], rows, g).wait()
            pltpu.make_async_copy(rec_hbm.at[dst_t.at[0]], rec_g, g).wait()
            pltpu.make_async_copy(lg_hbm.at[0, 0], lg_c, g).wait()

        def compute(c, b):
            rows, rec_g, lg_c, src_c, g = bufs[b]
            for g2 in range(CG // 16):
                sl = pl.ds(g2 * 16, 16)
                a16 = jnp.exp(lg_c[sl] - mvec) * rec_g[sl]
                for i in range(16):
                    r = g2 * 16 + i
                    bc = _bcast_lane(a16, i)
                    for d8 in range(D // 16):
                        sl2 = pl.ds(d8 * 16, 16)
                        rows[r, sl2] = rows[r, sl2] * bc
            pltpu.sync_copy(rows, acc_sh.at[dst_t.at[c]], add=True)

        issue(0, 0)

        def body(j, cy):
            c0 = 2 * j
            issue(c0 + 1, 1)
            wait_g(0)
            compute(c0, 0)
            issue(c0 + 2, 0)
            wait_g(1)
            compute(c0 + 1, 1)
            return cy

        lax.fori_loop(0, (NKG - 1) // 2, body, 0)
        wait_g(0)
        compute(NKG - 1, 0)

        plsc.subcore_barrier()
        pltpu.sync_copy(acc_sh.at[pl.ds(sid * SL, SL)],
                        out_hbm.at[cid, pl.ds(sid * SL, SL)])

    return k(logits, src, dst, mvec, rec, hn)


def _edge1_body(ef_ref, gsum_ref, w1_ref, wp_ref, attn_ref, logits_ref,
                f2pre_ref, mx_ref):
    i = pl.program_id(0)

    @pl.when(i == 0)
    def _init():
        mx_ref[...] = jnp.full((1, D), -1e30, jnp.float32)

    ef = ef_ref[...]
    f1 = jnp.dot(ef, w1_ref[...], preferred_element_type=jnp.float32)
    u = f1 + gsum_ref[...]
    fo = jnp.where(u > 0, u, 0.01 * u)
    lg = jnp.sum(fo * attn_ref[...], axis=1)
    logits_ref[...] = lg.reshape(1, 1, BE)
    cur = jnp.broadcast_to(jnp.max(lg), (1, D))
    mx_ref[...] = jnp.maximum(mx_ref[...], cur)
    h1 = jnp.maximum(fo, 0.0) + ef
    f2pre_ref[...] = jnp.dot(h1, wp_ref[...], preferred_element_type=jnp.float32)


def _edge1(ef, gsum, w1, wp, attn):
    logits3, f2pre, mx = pl.pallas_call(
        _edge1_body,
        grid=(NB,),
        in_specs=[
            pl.BlockSpec((BE, D), lambda i: (i, 0)),
            pl.BlockSpec((BE, D), lambda i: (i, 0)),
            pl.BlockSpec((D, D), lambda i: (0, 0)),
            pl.BlockSpec((D, D), lambda i: (0, 0)),
            pl.BlockSpec((1, D), lambda i: (0, 0)),
        ],
        out_specs=[
            pl.BlockSpec((1, 1, BE), lambda i: (i, 0, 0)),
            pl.BlockSpec((BE, D), lambda i: (i, 0)),
            pl.BlockSpec((1, D), lambda i: (0, 0)),
        ],
        out_shape=[
            jax.ShapeDtypeStruct((NB, 1, BE), jnp.float32),
            jax.ShapeDtypeStruct((E, D), jnp.float32),
            jax.ShapeDtypeStruct((1, D), jnp.float32),
        ],
    )(ef, gsum, w1, wp, attn)
    return logits3.reshape(E), f2pre, mx


def _edge2_body(f2pre_ref, gsum_ref, attn_ref, logits_ref, mx_ref):
    i = pl.program_id(0)

    @pl.when(i == 0)
    def _init():
        mx_ref[...] = jnp.full((1, D), -1e30, jnp.float32)

    u = f2pre_ref[...] + gsum_ref[...]
    fo = jnp.where(u > 0, u, 0.01 * u)
    lg = jnp.sum(fo * attn_ref[...], axis=1)
    logits_ref[...] = lg.reshape(1, 1, BE)
    cur = jnp.broadcast_to(jnp.max(lg), (1, D))
    mx_ref[...] = jnp.maximum(mx_ref[...], cur)


def _edge2(f2pre, gsum, attn):
    logits3, mx = pl.pallas_call(
        _edge2_body,
        grid=(NB,),
        in_specs=[
            pl.BlockSpec((BE, D), lambda i: (i, 0)),
            pl.BlockSpec((BE, D), lambda i: (i, 0)),
            pl.BlockSpec((1, D), lambda i: (0, 0)),
        ],
        out_specs=[
            pl.BlockSpec((1, 1, BE), lambda i: (i, 0, 0)),
            pl.BlockSpec((1, D), lambda i: (0, 0)),
        ],
        out_shape=[
            jax.ShapeDtypeStruct((NB, 1, BE), jnp.float32),
            jax.ShapeDtypeStruct((1, D), jnp.float32),
        ],
    )(f2pre, gsum, attn)
    return logits3.reshape(E), mx


BRN = 1000
NBN = N // BRN


def _node1_body(x_ref, wni_ref, wnj_ref, wnode_ref, b_ref,
                fni_ref, fnj_ref, hn_ref):
    x = x_ref[...]
    fni_ref[...] = jnp.dot(x, wni_ref[...], preferred_element_type=jnp.float32)
    fnj_ref[...] = jnp.dot(x, wnj_ref[...], preferred_element_type=jnp.float32)
    hn_ref[...] = (jnp.dot(x, wnode_ref[...], preferred_element_type=jnp.float32)
                   + b_ref[...])


def _node_tables1(x, wni, wnj, wnode, b):
    return pl.pallas_call(
        _node1_body,
        grid=(NBN,),
        in_specs=[
            pl.BlockSpec((BRN, D), lambda i: (i, 0)),
            pl.BlockSpec((D, D), lambda i: (0, 0)),
            pl.BlockSpec((D, D), lambda i: (0, 0)),
            pl.BlockSpec((D, D), lambda i: (0, 0)),
            pl.BlockSpec((1, D), lambda i: (0, 0)),
        ],
        out_specs=[
            pl.BlockSpec((BRN, D), lambda i: (i, 0)),
            pl.BlockSpec((BRN, D), lambda i: (i, 0)),
            pl.BlockSpec((BRN, D), lambda i: (i, 0)),
        ],
        out_shape=[jax.ShapeDtypeStruct((N, D), jnp.float32)] * 3,
    )(x, wni, wnj, wnode, b.reshape(1, D))


def _node2_body(accp_ref, x_ref, wni_ref, wnj_ref, wnode_ref, b_ref,
                fni_ref, fnj_ref, hn_ref):
    out1 = accp_ref[0] + accp_ref[1]
    h0 = jnp.maximum(out1, 0.0) + x_ref[...]
    fni_ref[...] = jnp.dot(h0, wni_ref[...], preferred_element_type=jnp.float32)
    fnj_ref[...] = jnp.dot(h0, wnj_ref[...], preferred_element_type=jnp.float32)
    hn_ref[...] = (jnp.dot(h0, wnode_ref[...], preferred_element_type=jnp.float32)
                   + b_ref[...])


def _node_tables2(accp, x, wni, wnj, wnode, b):
    return pl.pallas_call(
        _node2_body,
        grid=(NBN,),
        in_specs=[
            pl.BlockSpec((NC, BRN, D), lambda i: (0, i, 0)),
            pl.BlockSpec((BRN, D), lambda i: (i, 0)),
            pl.BlockSpec((D, D), lambda i: (0, 0)),
            pl.BlockSpec((D, D), lambda i: (0, 0)),
            pl.BlockSpec((D, D), lambda i: (0, 0)),
            pl.BlockSpec((1, D), lambda i: (0, 0)),
        ],
        out_specs=[
            pl.BlockSpec((BRN, D), lambda i: (i, 0)),
            pl.BlockSpec((BRN, D), lambda i: (i, 0)),
            pl.BlockSpec((BRN, D), lambda i: (i, 0)),
        ],
        out_shape=[jax.ShapeDtypeStruct((N, D), jnp.float32)] * 3,
    )(accp, x, wni, wnj, wnode, b.reshape(1, D))


def _final_body(accp_ref, x_ref, out_ref):
    out_ref[...] = accp_ref[0] + accp_ref[1] + x_ref[...]


def _final(accp, x):
    return pl.pallas_call(
        _final_body,
        grid=(NBN,),
        in_specs=[
            pl.BlockSpec((NC, BRN, D), lambda i: (0, i, 0)),
            pl.BlockSpec((BRN, D), lambda i: (i, 0)),
        ],
        out_specs=pl.BlockSpec((BRN, D), lambda i: (i, 0)),
        out_shape=jax.ShapeDtypeStruct((N, D), jnp.float32),
    )(accp, x)


def _egat_sparse(logits, mx, src3d32, dst3d32, dst3d16, hn):
    mvec = mx[0, :16]
    denp = _denom_sc(logits.reshape(NS, NKD, CG), dst3d16, mvec)
    rec = 1.0 / denp[0]
    return _agg_sc(logits.reshape(NW, NKG, CG), src3d32, dst3d32, mvec, rec,
                   hn)


def kernel(node_feats, edge_feats, edge_index, W_ni1, W_nj1, W_fij1, attn1,
           W_node1, b_node1, W_ni2, W_nj2, W_fij2, attn2, W_node2, b_node2):
    x = node_feats
    ef = edge_feats
    src3d32 = edge_index[0].reshape(NW, NKG, CG)
    dst3d32 = edge_index[1].reshape(NW, NKG, CG)
    dst3d16 = edge_index[1].reshape(NS, NKD, CG)

    f_ni1, f_nj1, hn1 = _node_tables1(x, W_ni1, W_nj1, W_node1, b_node1)
    gsum1 = _gsum_sc(f_ni1, f_nj1, src3d32, dst3d32)
    logits1, f2pre, mx1 = _edge1(ef, gsum1, W_fij1, W_fij2, attn1)
    accp1 = _egat_sparse(logits1, mx1, src3d32, dst3d32, dst3d16, hn1)

    f_ni2, f_nj2, hn2 = _node_tables2(accp1, x, W_ni2, W_nj2, W_node2, b_node2)
    gsum2 = _gsum_sc(f_ni2, f_nj2, src3d32, dst3d32)
    logits2, mx2 = _edge2(f2pre, gsum2, attn2)
    accp2 = _egat_sparse(logits2, mx2, src3d32, dst3d32, dst3d16, hn2)

    return _final(accp2, x)
